# baseline (speedup 1.0000x reference)
import functools

import jax
import jax.numpy as jnp
from jax import lax
from jax.experimental import pallas as pl
from jax.experimental.pallas import tpu as pltpu
from jax.experimental.pallas import tpu_sc as plsc

_MAXDET = 100
_NMS_T = 0.5
_THRES = 0.001
_NEG = -1e30
_NP = 1024
_CP = 96
_ND = _NP // 16
_SH = 24


def _sc_nms(logits_flat, boxes_flat, ts_pad, B):
    mesh = plsc.VectorSubcoreMesh(core_axis_name="c", subcore_axis_name="s")

    @functools.partial(
        pl.kernel,
        out_type=[
            jax.ShapeDtypeStruct((B, 128), jnp.float32),
            jax.ShapeDtypeStruct((B, 128), jnp.int32),
            jax.ShapeDtypeStruct((B, 512), jnp.float32),
            jax.ShapeDtypeStruct((B, 128), jnp.int32),
        ],
        mesh=mesh,
        scratch_types=[
            pltpu.VMEM((_CP * _NP,), jnp.float32),
            pltpu.VMEM((5 * _NP + 16,), jnp.float32),
            pltpu.VMEM((4 * _NP,), jnp.float32),
            pltpu.VMEM((_CP,), jnp.float32),
            pltpu.VMEM((_CP,), jnp.int32),
            pltpu.VMEM((128,), jnp.float32),
            pltpu.VMEM((128,), jnp.int32),
            pltpu.VMEM((512,), jnp.float32),
            pltpu.VMEM((128,), jnp.int32),
            pltpu.VMEM((16,), jnp.int32),
            pltpu.VMEM((32,), jnp.float32),
            pltpu.VMEM((32,), jnp.int32),
            pltpu.VMEM((32,), jnp.float32),
            pltpu.VMEM((32,), jnp.int32),
            pltpu.VMEM((128,), jnp.float32),
            pltpu.VMEM((128,), jnp.int32),
            pltpu.VMEM_SHARED((4, 3 * _SH * _NP), jnp.float32),
            pltpu.VMEM_SHARED((4, 128), jnp.float32),
            pltpu.VMEM_SHARED((4, 128), jnp.int32),
        ],
    )
    def k(logits_hbm, boxes_hbm, ts_hbm, osc_hbm, olb_hbm, obx_hbm, ovd_hbm,
          s_v, geom_v, bx_v, rmax_v, rarg_v, osc_v, olb_v, obx_v, ovd_v, ts_v,
          red_f, red_i, lrm_v, lra_v, tmpf, tmpi, slab, rm_sh, ra_sh):
        sidx = lax.axis_index("s")
        cidx = lax.axis_index("c")
        li = sidx // 4
        q = sidx - li * 4
        img = cidx * 4 + li
        sbase = q * _SH * _NP

        iota = lax.iota(jnp.int32, 16)
        neg16 = jnp.full((16,), _NEG, jnp.float32)
        zero16i = jnp.zeros((16,), jnp.int32)
        bigi = jnp.full((16,), 1 << 30, jnp.int32)

        def blend_store(ref, idx, val):
            base = (idx // 16) * 16
            lane = idx - base
            old = ref[pl.ds(base, 16)]
            ref[pl.ds(base, 16)] = jnp.where(iota == lane, val, old)

        red_f[pl.ds(16, 16)] = neg16
        red_i[pl.ds(16, 16)] = bigi

        def hargmax_pair(vals, keys):
            red_f[pl.ds(0, 16)] = vals
            red_i[pl.ds(0, 16)] = keys
            for sh in (8, 4, 2, 1):
                a = red_f[pl.ds(0, 16)]
                b = red_f[pl.ds(sh, 16)]
                ka = red_i[pl.ds(0, 16)]
                kb = red_i[pl.ds(sh, 16)]
                gt = a > b
                eq = a == b
                red_f[pl.ds(0, 16)] = jnp.maximum(a, b)
                red_i[pl.ds(0, 16)] = jnp.where(
                    gt, ka, jnp.where(eq, jnp.minimum(ka, kb), kb))
            return red_f[pl.ds(0, 16)][0], red_i[pl.ds(0, 16)][0]

        pltpu.sync_copy(logits_hbm.at[img, pl.ds(sbase, _SH * _NP)],
                        s_v.at[pl.ds(0, _SH * _NP)])

        def row_body(c, _):
            rb = c * _NP

            @plsc.parallel_loop(0, _ND // 8, carry=(neg16, zero16i))
            def init_carry(g, carry):
                vmax, varg = carry
                parts = []
                for u in range(8):
                    kk = g * 8 + u
                    x = s_v[pl.ds(rb + kk * 16, 16)]
                    sv = 1.0 / (1.0 + jnp.exp(-x))
                    parts.append((kk, sv, iota + kk * 16))
                for kk, sv, _ in parts:
                    s_v[pl.ds(rb + kk * 16, 16)] = sv
                for _, sv, nvec in parts:
                    m = sv > vmax
                    vmax = jnp.where(m, sv, vmax)
                    varg = jnp.where(m, nvec, varg)
                return vmax, varg

            mrow, nrow = hargmax_pair(*init_carry)
            blend_store(lrm_v, c, mrow)
            blend_store(lra_v, c, nrow)
            return 0

        lax.fori_loop(0, _SH, row_body, 0)

        @pl.when(q > 0)
        def _():
            pltpu.sync_copy(s_v.at[pl.ds(0, _SH * _NP)],
                            slab.at[li, pl.ds(sbase - _SH * _NP, _SH * _NP)])

        pltpu.sync_copy(lrm_v, rm_sh.at[li, pl.ds(q * 32, 32)])
        pltpu.sync_copy(lra_v, ra_sh.at[li, pl.ds(q * 32, 32)])

        plsc.subcore_barrier()

        @pl.when(q == 0)
        def _():
            pltpu.sync_copy(slab.at[li],
                            s_v.at[pl.ds(_SH * _NP, 3 * _SH * _NP)])
            pltpu.sync_copy(rm_sh.at[li], tmpf)
            pltpu.sync_copy(ra_sh.at[li], tmpi)
            pltpu.sync_copy(boxes_hbm.at[img], bx_v)
            pltpu.sync_copy(ts_hbm.at[img], ts_v)

            for j in range(_CP // 16):
                cv = iota + 16 * j
                mv = tmpf[pl.ds(16 * j, 16)]
                av = tmpi[pl.ds(16 * j, 16)]
                for qq in (1, 2, 3):
                    sel = cv >= qq * _SH
                    mv = jnp.where(sel, tmpf[pl.ds(8 * qq + 16 * j, 16)], mv)
                    av = jnp.where(sel, tmpi[pl.ds(8 * qq + 16 * j, 16)], av)
                rmax_v[pl.ds(16 * j, 16)] = mv
                rarg_v[pl.ds(16 * j, 16)] = av

            @plsc.parallel_loop(0, _ND, unroll=4)
            def geom_body(kk):
                o = kk * 16
                cx = bx_v[pl.ds(o, 16)]
                cy = bx_v[pl.ds(_NP + o, 16)]
                w = bx_v[pl.ds(2 * _NP + o, 16)]
                h = bx_v[pl.ds(3 * _NP + o, 16)]
                x0 = cx - 0.5 * w
                y0 = cy - 0.5 * h
                x1 = cx + 0.5 * w
                y1 = cy + 0.5 * h
                geom_v[pl.ds(o, 16)] = x0
                geom_v[pl.ds(_NP + o, 16)] = y0
                geom_v[pl.ds(2 * _NP + o, 16)] = x1
                geom_v[pl.ds(3 * _NP + o, 16)] = y1
                geom_v[pl.ds(4 * _NP + o, 16)] = (x1 - x0) * (y1 - y0)

            @plsc.parallel_loop(0, 8)
            def zo_body(kk):
                o = kk * 16
                osc_v[pl.ds(o, 16)] = jnp.zeros((16,), jnp.float32)
                olb_v[pl.ds(o, 16)] = zero16i
                ovd_v[pl.ds(o, 16)] = zero16i

            @plsc.parallel_loop(0, 32)
            def zb_body(kk):
                obx_v[pl.ds(kk * 16, 16)] = jnp.zeros((16,), jnp.float32)

            tsvec = ts_v[pl.ds(0, 16)]
            hf = tsvec[0].astype(jnp.float32)
            wf = tsvec[1].astype(jnp.float32)

            def it_body(i, _):
                @plsc.parallel_loop(0, _CP // 16, carry=(neg16, bigi))
                def mx_body(kk, carry):
                    vmax, vkey = carry
                    rm = rmax_v[pl.ds(kk * 16, 16)]
                    ra = rarg_v[pl.ds(kk * 16, 16)]
                    key = ra * 128 + (iota + kk * 16)
                    gt = rm > vmax
                    eq = rm == vmax
                    nkey = jnp.where(gt, key, jnp.where(eq, jnp.minimum(vkey, key), vkey))
                    return jnp.maximum(rm, vmax), nkey

                mglob, j2 = hargmax_pair(*mx_body)
                ok = mglob > _THRES

                @pl.when(ok)
                def _():
                    n = j2 // 128
                    c = j2 - n * 128
                    x0b = geom_v[pl.ds(n, 16)][0]
                    y0b = geom_v[pl.ds(_NP + n, 16)][0]
                    x1b = geom_v[pl.ds(2 * _NP + n, 16)][0]
                    y1b = geom_v[pl.ds(3 * _NP + n, 16)][0]
                    areab = (x1b - x0b) * (y1b - y0b)
                    rb = c * _NP
                    blend_store(s_v, rb + n, _NEG)

                    @plsc.parallel_loop(0, _ND // 8, carry=(neg16, zero16i))
                    def upd_body(g, carry):
                        vmax, varg = carry
                        parts = []
                        for u in range(8):
                            kk = g * 8 + u
                            o = kk * 16
                            x0 = geom_v[pl.ds(o, 16)]
                            y0 = geom_v[pl.ds(_NP + o, 16)]
                            x1 = geom_v[pl.ds(2 * _NP + o, 16)]
                            y1 = geom_v[pl.ds(3 * _NP + o, 16)]
                            ar = geom_v[pl.ds(4 * _NP + o, 16)]
                            sc = s_v[pl.ds(rb + o, 16)]
                            inter = jnp.maximum(jnp.minimum(x1, x1b) - jnp.maximum(x0, x0b), 0.0)
                            inter = inter * jnp.maximum(jnp.minimum(y1, y1b) - jnp.maximum(y0, y0b), 0.0)
                            iou = inter / jnp.maximum(areab + ar - inter, 1e-9)
                            nvec = iota + o
                            ns = jnp.where(iou > _NMS_T, _NEG, sc)
                            parts.append((kk, ns, nvec))
                        for kk, ns, _ in parts:
                            s_v[pl.ds(rb + kk * 16, 16)] = ns
                        for _, ns, nvec in parts:
                            m = ns > vmax
                            vmax = jnp.where(m, ns, vmax)
                            varg = jnp.where(m, nvec, varg)
                        return vmax, varg

                    mrow, nrow = hargmax_pair(*upd_body)
                    blend_store(rmax_v, c, mrow)
                    blend_store(rarg_v, c, nrow)
                    blend_store(osc_v, i, mglob)
                    blend_store(olb_v, i, c)
                    blend_store(ovd_v, i, jnp.int32(1))
                    pos = 4 * i
                    bbase = (pos // 16) * 16
                    l0 = pos - bbase
                    old = obx_v[pl.ds(bbase, 16)]
                    bv = jnp.where(iota == l0, x0b * wf, old)
                    bv = jnp.where(iota == l0 + 1, y0b * hf, bv)
                    bv = jnp.where(iota == l0 + 2, x1b * wf, bv)
                    bv = jnp.where(iota == l0 + 3, y1b * hf, bv)
                    obx_v[pl.ds(bbase, 16)] = bv

                return 0

            lax.fori_loop(0, _MAXDET, it_body, 0)

            pltpu.sync_copy(osc_v, osc_hbm.at[img])
            pltpu.sync_copy(olb_v, olb_hbm.at[img])
            pltpu.sync_copy(obx_v, obx_hbm.at[img])
            pltpu.sync_copy(ovd_v, ovd_hbm.at[img])

    return k(logits_flat, boxes_flat, ts_pad)


def kernel(pred_logits, pred_boxes, target_sizes):
    B, N, C = pred_logits.shape
    lt = jnp.transpose(pred_logits, (0, 2, 1))
    lt = jnp.pad(lt, ((0, 0), (0, _CP - C), (0, _NP - N)),
                 constant_values=-1e9)
    lflat = lt.reshape(B, _CP * _NP)
    bt = jnp.transpose(pred_boxes, (0, 2, 1))
    bt = jnp.pad(bt, ((0, 0), (0, 0), (0, _NP - N)))
    bflat = bt.reshape(B, 4 * _NP)
    tsp = jnp.pad(target_sizes, ((0, 0), (0, 16 - target_sizes.shape[1])))
    osc, olb, obx, ovd = _sc_nms(lflat, bflat, tsp, B)
    return (
        osc[:, :_MAXDET],
        olb[:, :_MAXDET],
        obx.reshape(B, 128, 4)[:, :_MAXDET, :],
        ovd[:, :_MAXDET] != 0,
    )


# speedup vs baseline: 1.0066x; 1.0066x over previous
import functools

import jax
import jax.numpy as jnp
from jax import lax
from jax.experimental import pallas as pl
from jax.experimental.pallas import tpu as pltpu
from jax.experimental.pallas import tpu_sc as plsc

_MAXDET = 100
_NMS_T = 0.5
_THRES = 0.001
_NEG = -1e30
_NP = 1024
_CP = 96
_ND = _NP // 16
_SH = 24


def _sc_nms(logits_flat, boxes_flat, ts_pad, B):
    mesh = plsc.VectorSubcoreMesh(core_axis_name="c", subcore_axis_name="s")

    @functools.partial(
        pl.kernel,
        out_type=[
            jax.ShapeDtypeStruct((B, 128), jnp.float32),
            jax.ShapeDtypeStruct((B, 128), jnp.int32),
            jax.ShapeDtypeStruct((B, 512), jnp.float32),
            jax.ShapeDtypeStruct((B, 128), jnp.int32),
        ],
        mesh=mesh,
        scratch_types=[
            pltpu.VMEM((_CP * _NP,), jnp.float32),
            pltpu.VMEM((4 * _NP + 16,), jnp.float32),
            pltpu.VMEM((4 * _NP,), jnp.float32),
            pltpu.VMEM((_CP,), jnp.float32),
            pltpu.VMEM((_CP,), jnp.int32),
            pltpu.VMEM((128,), jnp.float32),
            pltpu.VMEM((128,), jnp.int32),
            pltpu.VMEM((512,), jnp.float32),
            pltpu.VMEM((128,), jnp.int32),
            pltpu.VMEM((16,), jnp.int32),
            pltpu.VMEM((32,), jnp.float32),
            pltpu.VMEM((32,), jnp.int32),
            pltpu.VMEM((32,), jnp.float32),
            pltpu.VMEM((32,), jnp.int32),
            pltpu.VMEM((128,), jnp.float32),
            pltpu.VMEM((128,), jnp.int32),
            pltpu.VMEM_SHARED((4, 3 * _SH * _NP), jnp.float32),
            pltpu.VMEM_SHARED((4, 128), jnp.float32),
            pltpu.VMEM_SHARED((4, 128), jnp.int32),
        ],
    )
    def k(logits_hbm, boxes_hbm, ts_hbm, osc_hbm, olb_hbm, obx_hbm, ovd_hbm,
          s_v, geom_v, bx_v, rmax_v, rarg_v, osc_v, olb_v, obx_v, ovd_v, ts_v,
          red_f, red_i, lrm_v, lra_v, tmpf, tmpi, slab, rm_sh, ra_sh):
        sidx = lax.axis_index("s")
        cidx = lax.axis_index("c")
        li = sidx // 4
        q = sidx - li * 4
        img = cidx * 4 + li
        sbase = q * _SH * _NP

        iota = lax.iota(jnp.int32, 16)
        neg16 = jnp.full((16,), _NEG, jnp.float32)
        zero16i = jnp.zeros((16,), jnp.int32)
        bigi = jnp.full((16,), 1 << 30, jnp.int32)

        def blend_store(ref, idx, val):
            base = (idx // 16) * 16
            lane = idx - base
            old = ref[pl.ds(base, 16)]
            ref[pl.ds(base, 16)] = jnp.where(iota == lane, val, old)

        red_f[pl.ds(16, 16)] = neg16
        red_i[pl.ds(16, 16)] = bigi

        def hargmax_pair(vals, keys):
            red_f[pl.ds(0, 16)] = vals
            red_i[pl.ds(0, 16)] = keys
            for sh in (8, 4, 2, 1):
                a = red_f[pl.ds(0, 16)]
                b = red_f[pl.ds(sh, 16)]
                ka = red_i[pl.ds(0, 16)]
                kb = red_i[pl.ds(sh, 16)]
                gt = a > b
                eq = a == b
                red_f[pl.ds(0, 16)] = jnp.maximum(a, b)
                red_i[pl.ds(0, 16)] = jnp.where(
                    gt, ka, jnp.where(eq, jnp.minimum(ka, kb), kb))
            return red_f[pl.ds(0, 16)][0], red_i[pl.ds(0, 16)][0]

        pltpu.sync_copy(logits_hbm.at[img, pl.ds(sbase, _SH * _NP)],
                        s_v.at[pl.ds(0, _SH * _NP)])

        def row_body(c, _):
            rb = c * _NP

            @plsc.parallel_loop(0, _ND // 8, carry=(neg16, zero16i))
            def init_carry(g, carry):
                vmax, varg = carry
                parts = []
                for u in range(8):
                    kk = g * 8 + u
                    x = s_v[pl.ds(rb + kk * 16, 16)]
                    sv = 1.0 / (1.0 + jnp.exp(-x))
                    parts.append((kk, sv, iota + kk * 16))
                for kk, sv, _ in parts:
                    s_v[pl.ds(rb + kk * 16, 16)] = sv
                for _, sv, nvec in parts:
                    m = sv > vmax
                    vmax = jnp.where(m, sv, vmax)
                    varg = jnp.where(m, nvec, varg)
                return vmax, varg

            mrow, nrow = hargmax_pair(*init_carry)
            blend_store(lrm_v, c, mrow)
            blend_store(lra_v, c, nrow)
            return 0

        lax.fori_loop(0, _SH, row_body, 0)

        @pl.when(q > 0)
        def _():
            pltpu.sync_copy(s_v.at[pl.ds(0, _SH * _NP)],
                            slab.at[li, pl.ds(sbase - _SH * _NP, _SH * _NP)])

        pltpu.sync_copy(lrm_v, rm_sh.at[li, pl.ds(q * 32, 32)])
        pltpu.sync_copy(lra_v, ra_sh.at[li, pl.ds(q * 32, 32)])

        plsc.subcore_barrier()

        @pl.when(q == 0)
        def _():
            pltpu.sync_copy(slab.at[li],
                            s_v.at[pl.ds(_SH * _NP, 3 * _SH * _NP)])
            pltpu.sync_copy(rm_sh.at[li], tmpf)
            pltpu.sync_copy(ra_sh.at[li], tmpi)
            pltpu.sync_copy(boxes_hbm.at[img], bx_v)
            pltpu.sync_copy(ts_hbm.at[img], ts_v)

            for j in range(_CP // 16):
                cv = iota + 16 * j
                mv = tmpf[pl.ds(16 * j, 16)]
                av = tmpi[pl.ds(16 * j, 16)]
                for qq in (1, 2, 3):
                    sel = cv >= qq * _SH
                    mv = jnp.where(sel, tmpf[pl.ds(8 * qq + 16 * j, 16)], mv)
                    av = jnp.where(sel, tmpi[pl.ds(8 * qq + 16 * j, 16)], av)
                rmax_v[pl.ds(16 * j, 16)] = mv
                rarg_v[pl.ds(16 * j, 16)] = av

            @plsc.parallel_loop(0, _ND, unroll=4)
            def geom_body(kk):
                o = kk * 16
                cx = bx_v[pl.ds(o, 16)]
                cy = bx_v[pl.ds(_NP + o, 16)]
                w = bx_v[pl.ds(2 * _NP + o, 16)]
                h = bx_v[pl.ds(3 * _NP + o, 16)]
                x0 = cx - 0.5 * w
                y0 = cy - 0.5 * h
                x1 = cx + 0.5 * w
                y1 = cy + 0.5 * h
                geom_v[pl.ds(o, 16)] = x0
                geom_v[pl.ds(_NP + o, 16)] = y0
                geom_v[pl.ds(2 * _NP + o, 16)] = x1
                geom_v[pl.ds(3 * _NP + o, 16)] = y1

            @plsc.parallel_loop(0, 8)
            def zo_body(kk):
                o = kk * 16
                osc_v[pl.ds(o, 16)] = jnp.zeros((16,), jnp.float32)
                olb_v[pl.ds(o, 16)] = zero16i
                ovd_v[pl.ds(o, 16)] = zero16i

            @plsc.parallel_loop(0, 32)
            def zb_body(kk):
                obx_v[pl.ds(kk * 16, 16)] = jnp.zeros((16,), jnp.float32)

            tsvec = ts_v[pl.ds(0, 16)]
            hf = tsvec[0].astype(jnp.float32)
            wf = tsvec[1].astype(jnp.float32)

            def it_body(i, _):
                @plsc.parallel_loop(0, _CP // 16, carry=(neg16, bigi))
                def mx_body(kk, carry):
                    vmax, vkey = carry
                    rm = rmax_v[pl.ds(kk * 16, 16)]
                    ra = rarg_v[pl.ds(kk * 16, 16)]
                    key = ra * 128 + (iota + kk * 16)
                    gt = rm > vmax
                    eq = rm == vmax
                    nkey = jnp.where(gt, key, jnp.where(eq, jnp.minimum(vkey, key), vkey))
                    return jnp.maximum(rm, vmax), nkey

                mglob, j2 = hargmax_pair(*mx_body)
                ok = mglob > _THRES

                @pl.when(ok)
                def _():
                    n = j2 // 128
                    c = j2 - n * 128
                    x0b = geom_v[pl.ds(n, 16)][0]
                    y0b = geom_v[pl.ds(_NP + n, 16)][0]
                    x1b = geom_v[pl.ds(2 * _NP + n, 16)][0]
                    y1b = geom_v[pl.ds(3 * _NP + n, 16)][0]
                    areab = (x1b - x0b) * (y1b - y0b)
                    rb = c * _NP

                    @plsc.parallel_loop(0, _ND // 8, carry=(neg16, zero16i))
                    def upd_body(g, carry):
                        vmax, varg = carry
                        parts = []
                        for u in range(8):
                            kk = g * 8 + u
                            o = kk * 16
                            x0 = geom_v[pl.ds(o, 16)]
                            y0 = geom_v[pl.ds(_NP + o, 16)]
                            x1 = geom_v[pl.ds(2 * _NP + o, 16)]
                            y1 = geom_v[pl.ds(3 * _NP + o, 16)]
                            ar = (x1 - x0) * (y1 - y0)
                            sc = s_v[pl.ds(rb + o, 16)]
                            inter = jnp.maximum(jnp.minimum(x1, x1b) - jnp.maximum(x0, x0b), 0.0)
                            inter = inter * jnp.maximum(jnp.minimum(y1, y1b) - jnp.maximum(y0, y0b), 0.0)
                            iou = inter / jnp.maximum(areab + ar - inter, 1e-9)
                            nvec = iota + o
                            ns = jnp.where((iou > _NMS_T) | (nvec == n), _NEG, sc)
                            parts.append((kk, ns, nvec))
                        for kk, ns, _ in parts:
                            s_v[pl.ds(rb + kk * 16, 16)] = ns
                        for _, ns, nvec in parts:
                            m = ns > vmax
                            vmax = jnp.where(m, ns, vmax)
                            varg = jnp.where(m, nvec, varg)
                        return vmax, varg

                    mrow, nrow = hargmax_pair(*upd_body)
                    blend_store(rmax_v, c, mrow)
                    blend_store(rarg_v, c, nrow)
                    blend_store(osc_v, i, mglob)
                    blend_store(olb_v, i, c)
                    blend_store(ovd_v, i, jnp.int32(1))
                    pos = 4 * i
                    bbase = (pos // 16) * 16
                    l0 = pos - bbase
                    old = obx_v[pl.ds(bbase, 16)]
                    bv = jnp.where(iota == l0, x0b * wf, old)
                    bv = jnp.where(iota == l0 + 1, y0b * hf, bv)
                    bv = jnp.where(iota == l0 + 2, x1b * wf, bv)
                    bv = jnp.where(iota == l0 + 3, y1b * hf, bv)
                    obx_v[pl.ds(bbase, 16)] = bv

                return 0

            lax.fori_loop(0, _MAXDET, it_body, 0)

            pltpu.sync_copy(osc_v, osc_hbm.at[img])
            pltpu.sync_copy(olb_v, olb_hbm.at[img])
            pltpu.sync_copy(obx_v, obx_hbm.at[img])
            pltpu.sync_copy(ovd_v, ovd_hbm.at[img])

    return k(logits_flat, boxes_flat, ts_pad)


def kernel(pred_logits, pred_boxes, target_sizes):
    B, N, C = pred_logits.shape
    lt = jnp.transpose(pred_logits, (0, 2, 1))
    lt = jnp.pad(lt, ((0, 0), (0, _CP - C), (0, _NP - N)),
                 constant_values=-1e9)
    lflat = lt.reshape(B, _CP * _NP)
    bt = jnp.transpose(pred_boxes, (0, 2, 1))
    bt = jnp.pad(bt, ((0, 0), (0, 0), (0, _NP - N)))
    bflat = bt.reshape(B, 4 * _NP)
    tsp = jnp.pad(target_sizes, ((0, 0), (0, 16 - target_sizes.shape[1])))
    osc, olb, obx, ovd = _sc_nms(lflat, bflat, tsp, B)
    return (
        osc[:, :_MAXDET],
        olb[:, :_MAXDET],
        obx.reshape(B, 128, 4)[:, :_MAXDET, :],
        ovd[:, :_MAXDET] != 0,
    )
